# baseline (speedup 1.0000x reference)
import functools
import jax
import jax.numpy as jnp
from jax import lax
from jax.experimental import pallas as pl
from jax.experimental.pallas import tpu as pltpu
from jax.experimental.pallas import tpu_sc as plsc

N = 10000
E = 320000
D = 128
DG = 16
NC, NS = 2, 16
NW = NC * NS
K = 40
CHUNKS = 250
NACC = 10112
ROWS_PER_TILE = NACC // NS


def _sc_aggregate(x, edges, zrows, zdeg):
    mesh = plsc.VectorSubcoreMesh(core_axis_name="c", subcore_axis_name="s")

    @functools.partial(
        pl.kernel,
        out_type=(
            jax.ShapeDtypeStruct((NC, NACC, D), jnp.float32),
            jax.ShapeDtypeStruct((NC, NACC, DG), jnp.float32),
        ),
        mesh=mesh,
        scratch_types=[
            pltpu.VMEM((CHUNKS, K), jnp.int32),
            pltpu.VMEM((CHUNKS, K), jnp.int32),
            pltpu.VMEM((K, D), jnp.float32),
            pltpu.VMEM((K, D), jnp.float32),
            pltpu.VMEM((K, D), jnp.float32),
            pltpu.VMEM((K, DG), jnp.float32),
            pltpu.VMEM_SHARED((NACC, D), jnp.float32),
            pltpu.VMEM_SHARED((NACC, DG), jnp.float32),
            pltpu.SemaphoreType.DMA,
            pltpu.SemaphoreType.DMA,
            pltpu.SemaphoreType.DMA,
            pltpu.SemaphoreType.DMA,
            pltpu.SemaphoreType.DMA,
            pltpu.SemaphoreType.DMA,
            pltpu.SemaphoreType.DMA,
        ],
        compiler_params=pltpu.CompilerParams(use_tc_tiling_on_sc=False),
    )
    def body(x_ref, e_ref, zr_ref, zd_ref, out_ref, deg_ref,
             sidx, didx, rows0, rows1, rows2, ones, acc, dacc,
             gsem0, gsem1, gsem2, ssem0, ssem1, ssem2, xsem):
        c = lax.axis_index("c")
        s = lax.axis_index("s")
        w = s * NC + c

        rows = (rows0, rows1, rows2)
        gsem = (gsem0, gsem1, gsem2)
        ssem = (ssem0, ssem1, ssem2)

        def g_start(i, b):
            pltpu.make_async_copy(x_ref.at[sidx.at[i]], rows[b], gsem[b]).start()

        def g_wait(b):
            pltpu.make_async_copy(x_ref.at[sidx.at[0]], rows[b], gsem[b]).wait()

        def s_start(i, b):
            pltpu.make_async_copy(rows[b], acc.at[didx.at[i]], ssem[b]).start(add=True)
            pltpu.make_async_copy(ones, dacc.at[didx.at[i]], ssem[b]).start(add=True)

        def s_wait(b):
            pltpu.make_async_copy(rows[b], acc.at[didx.at[0]], ssem[b]).wait()
            pltpu.make_async_copy(ones, dacc.at[didx.at[0]], ssem[b]).wait()

        stripe = pl.ds(s * ROWS_PER_TILE, ROWS_PER_TILE)

        with jax.named_scope("stage_zero"):
            pltpu.make_async_copy(e_ref.at[0, w], sidx, xsem).start()
            pltpu.make_async_copy(e_ref.at[1, w], didx, xsem).start()
            pltpu.make_async_copy(zr_ref, acc.at[stripe], ssem0).start()
            pltpu.make_async_copy(zd_ref, dacc.at[stripe], ssem1).start()

            one = jnp.ones((16,), jnp.float32)
            for i in range(K):
                ones[i, :] = one

            pltpu.make_async_copy(e_ref.at[0, w], sidx, xsem).wait()
            pltpu.make_async_copy(e_ref.at[1, w], didx, xsem).wait()

        with jax.named_scope("edges"):
            g_start(0, 0)
            g_start(1, 1)
            pltpu.make_async_copy(zr_ref, acc.at[stripe], ssem0).wait()
            pltpu.make_async_copy(zd_ref, dacc.at[stripe], ssem1).wait()
            plsc.subcore_barrier()

            g_wait(0)
            s_start(0, 0)
            g_start(2, 2)

            def pos(i, b):
                g_wait(b)
                s_wait((b + 2) % 3)
                s_start(i, b)
                g_start(i + 2, (b + 2) % 3)

            def step(t, carry):
                i = 3 * t + 1
                pos(i, 1)
                pos(i + 1, 2)
                pos(i + 2, 0)
                return carry
            lax.fori_loop(0, (CHUNKS - 4) // 3, step, 0)

            g_wait(1)
            s_wait(0)
            s_start(CHUNKS - 3, 1)
            g_start(CHUNKS - 1, 0)
            g_wait(2)
            s_wait(1)
            s_start(CHUNKS - 2, 2)
            g_wait(0)
            s_wait(2)
            s_start(CHUNKS - 1, 0)
            s_wait(0)
            plsc.subcore_barrier()

        with jax.named_scope("dump"):
            pltpu.sync_copy(acc.at[stripe], out_ref.at[c, stripe])
            pltpu.sync_copy(dacc.at[stripe], deg_ref.at[c, stripe])

    return body(x, edges, zrows, zdeg)


R = 1000


def _tc_self(x, W_self, bias):
    def body(x_ref, ws_ref, b_ref, o_ref):
        dn = (((1,), (1,)), ((), ()))
        o_ref[...] = lax.dot_general(
            x_ref[...], ws_ref[...], dn, preferred_element_type=jnp.float32
        ) + b_ref[...]

    return pl.pallas_call(
        body,
        grid=(N // R,),
        in_specs=[
            pl.BlockSpec((R, D), lambda i: (i, 0)),
            pl.BlockSpec((D, D), lambda i: (0, 0)),
            pl.BlockSpec((1, D), lambda i: (0, 0)),
        ],
        out_specs=pl.BlockSpec((R, D), lambda i: (i, 0)),
        out_shape=jax.ShapeDtypeStruct((N, D), jnp.float32),
    )(x, W_self, bias)


def _tc_finish(self_term, psum, dsum, W_neigh, gamma, beta):
    def body(st_ref, p_ref, d_ref, wn_ref, g_ref, be_ref, o_ref):
        p = p_ref[...]
        agg = p[0] + p[1]
        dg = d_ref[...]
        deg = jnp.maximum(dg[0, :, 0] + dg[1, :, 0], 1.0)
        neigh = agg / deg[:, None]
        dn = (((1,), (1,)), ((), ()))
        out = (st_ref[...]
               + lax.dot_general(neigh, wn_ref[...], dn, preferred_element_type=jnp.float32))
        out = jnp.maximum(out, 0.0)
        mu = jnp.mean(out, axis=-1, keepdims=True)
        var = jnp.mean((out - mu) ** 2, axis=-1, keepdims=True)
        o_ref[...] = ((out - mu) * lax.rsqrt(var + 1e-5)) * g_ref[...] + be_ref[...]

    return pl.pallas_call(
        body,
        grid=(N // R,),
        in_specs=[
            pl.BlockSpec((R, D), lambda i: (i, 0)),
            pl.BlockSpec((NC, R, D), lambda i: (0, i, 0)),
            pl.BlockSpec((NC, R, DG), lambda i: (0, i, 0)),
            pl.BlockSpec((D, D), lambda i: (0, 0)),
            pl.BlockSpec((1, D), lambda i: (0, 0)),
            pl.BlockSpec((1, D), lambda i: (0, 0)),
        ],
        out_specs=pl.BlockSpec((R, D), lambda i: (i, 0)),
        out_shape=jax.ShapeDtypeStruct((N, D), jnp.float32),
    )(self_term, psum, dsum, W_neigh, gamma, beta)


def kernel(x, edge_index, W_self, W_neigh, bias, ln_gamma, ln_beta):
    edges = edge_index.astype(jnp.int32).reshape(2, NW, CHUNKS, K)
    zrows = jnp.zeros((ROWS_PER_TILE, D), jnp.float32)
    zdeg = jnp.zeros((ROWS_PER_TILE, DG), jnp.float32)
    psum, dsum = _sc_aggregate(x, edges, zrows, zdeg)
    self_term = _tc_self(x, W_self, bias.reshape(1, D))
    return _tc_finish(
        self_term, psum, dsum, W_neigh,
        ln_gamma.reshape(1, D), ln_beta.reshape(1, D),
    )


# speedup vs baseline: 1.0134x; 1.0134x over previous
import functools
import jax
import jax.numpy as jnp
from jax import lax
from jax.experimental import pallas as pl
from jax.experimental.pallas import tpu as pltpu
from jax.experimental.pallas import tpu_sc as plsc

N = 10000
E = 320000
D = 128
DG = 16
NC, NS = 2, 16
NW = NC * NS
K = 40
CHUNKS = 250
NACC = 10112
ROWS_PER_TILE = NACC // NS


def _sc_aggregate(x, edges, zrows, zdeg):
    mesh = plsc.VectorSubcoreMesh(core_axis_name="c", subcore_axis_name="s")

    @functools.partial(
        pl.kernel,
        out_type=(
            jax.ShapeDtypeStruct((NC, NACC, D), jnp.float32),
            jax.ShapeDtypeStruct((NC, NACC, DG), jnp.float32),
        ),
        mesh=mesh,
        scratch_types=[
            pltpu.VMEM((CHUNKS, K), jnp.int32),
            pltpu.VMEM((CHUNKS, K), jnp.int32),
            pltpu.VMEM((K, D), jnp.float32),
            pltpu.VMEM((K, D), jnp.float32),
            pltpu.VMEM((K, D), jnp.float32),
            pltpu.VMEM((K, DG), jnp.float32),
            pltpu.VMEM_SHARED((NACC, D), jnp.float32),
            pltpu.VMEM_SHARED((NACC, DG), jnp.float32),
            pltpu.SemaphoreType.DMA,
            pltpu.SemaphoreType.DMA,
            pltpu.SemaphoreType.DMA,
            pltpu.SemaphoreType.DMA,
            pltpu.SemaphoreType.DMA,
            pltpu.SemaphoreType.DMA,
            pltpu.SemaphoreType.DMA,
        ],
        compiler_params=pltpu.CompilerParams(use_tc_tiling_on_sc=False),
    )
    def body(x_ref, e_ref, zr_ref, zd_ref, out_ref, deg_ref,
             sidx, didx, rows0, rows1, rows2, ones, acc, dacc,
             gsem0, gsem1, gsem2, ssem0, ssem1, ssem2, xsem):
        c = lax.axis_index("c")
        s = lax.axis_index("s")
        w = s * NC + c

        rows = (rows0, rows1, rows2)
        gsem = (gsem0, gsem1, gsem2)
        ssem = (ssem0, ssem1, ssem2)

        def g_start(i, b):
            pltpu.make_async_copy(x_ref.at[sidx.at[i]], rows[b], gsem[b]).start()

        def g_wait(b):
            pltpu.make_async_copy(x_ref.at[sidx.at[0]], rows[b], gsem[b]).wait()

        def s_start(i, b):
            pltpu.make_async_copy(rows[b], acc.at[didx.at[i]], ssem[b]).start(add=True)
            pltpu.make_async_copy(ones, dacc.at[didx.at[i]], ssem[b]).start(add=True)

        def s_wait(b):
            pltpu.make_async_copy(rows[b], acc.at[didx.at[0]], ssem[b]).wait()
            pltpu.make_async_copy(ones, dacc.at[didx.at[0]], ssem[b]).wait()

        stripe = pl.ds(s * ROWS_PER_TILE, ROWS_PER_TILE)

        with jax.named_scope("stage_zero"):
            pltpu.make_async_copy(e_ref.at[0, w], sidx, xsem).start()
            pltpu.make_async_copy(e_ref.at[1, w], didx, xsem).start()
            pltpu.make_async_copy(zr_ref, acc.at[stripe], ssem0).start()
            pltpu.make_async_copy(zd_ref, dacc.at[stripe], ssem1).start()

            one = jnp.ones((16,), jnp.float32)
            for i in range(K):
                ones[i, :] = one

            pltpu.make_async_copy(e_ref.at[0, w], sidx, xsem).wait()
            pltpu.make_async_copy(e_ref.at[1, w], didx, xsem).wait()

        with jax.named_scope("edges"):
            g_start(0, 0)
            g_start(1, 1)
            pltpu.make_async_copy(zr_ref, acc.at[stripe], ssem0).wait()
            pltpu.make_async_copy(zd_ref, dacc.at[stripe], ssem1).wait()
            plsc.subcore_barrier()

            g_wait(0)
            s_start(0, 0)
            g_start(2, 2)

            def pos(i, b):
                g_wait(b)
                s_wait((b + 2) % 3)
                s_start(i, b)
                g_start(i + 2, (b + 2) % 3)

            def step(t, carry):
                i = 3 * t + 1
                pos(i, 1)
                pos(i + 1, 2)
                pos(i + 2, 0)
                return carry
            lax.fori_loop(0, (CHUNKS - 4) // 3, step, 0)

            g_wait(1)
            s_wait(0)
            s_start(CHUNKS - 3, 1)
            g_start(CHUNKS - 1, 0)
            g_wait(2)
            s_wait(1)
            s_start(CHUNKS - 2, 2)
            g_wait(0)
            s_wait(2)
            s_start(CHUNKS - 1, 0)
            s_wait(0)
            plsc.subcore_barrier()

        with jax.named_scope("dump"):
            pltpu.sync_copy(acc.at[stripe], out_ref.at[c, stripe])
            pltpu.sync_copy(dacc.at[stripe], deg_ref.at[c, stripe])

    return body(x, edges, zrows, zdeg)


R = 1000


def _tc_finish(x, psum, dsum, W_self, W_neigh, bias, gamma, beta):
    def body(x_ref, p_ref, d_ref, ws_ref, wn_ref, b_ref, g_ref, be_ref, o_ref):
        p = p_ref[...]
        agg = p[0] + p[1]
        dg = d_ref[...]
        deg = jnp.maximum(dg[0, :, 0] + dg[1, :, 0], 1.0)
        neigh = agg / deg[:, None]
        xv = x_ref[...]
        dn = (((1,), (1,)), ((), ()))
        out = (lax.dot_general(xv, ws_ref[...], dn, preferred_element_type=jnp.float32)
               + lax.dot_general(neigh, wn_ref[...], dn, preferred_element_type=jnp.float32)
               + b_ref[...])
        out = jnp.maximum(out, 0.0)
        mu = jnp.mean(out, axis=-1, keepdims=True)
        var = jnp.mean((out - mu) ** 2, axis=-1, keepdims=True)
        o_ref[...] = ((out - mu) * lax.rsqrt(var + 1e-5)) * g_ref[...] + be_ref[...]

    return pl.pallas_call(
        body,
        grid=(N // R,),
        in_specs=[
            pl.BlockSpec((R, D), lambda i: (i, 0)),
            pl.BlockSpec((NC, R, D), lambda i: (0, i, 0)),
            pl.BlockSpec((NC, R, DG), lambda i: (0, i, 0)),
            pl.BlockSpec((D, D), lambda i: (0, 0)),
            pl.BlockSpec((D, D), lambda i: (0, 0)),
            pl.BlockSpec((1, D), lambda i: (0, 0)),
            pl.BlockSpec((1, D), lambda i: (0, 0)),
            pl.BlockSpec((1, D), lambda i: (0, 0)),
        ],
        out_specs=pl.BlockSpec((R, D), lambda i: (i, 0)),
        out_shape=jax.ShapeDtypeStruct((N, D), jnp.float32),
    )(x, psum, dsum, W_self, W_neigh, bias, gamma, beta)


def kernel(x, edge_index, W_self, W_neigh, bias, ln_gamma, ln_beta):
    edges = edge_index.astype(jnp.int32).reshape(2, NW, CHUNKS, K)
    zrows = jnp.zeros((ROWS_PER_TILE, D), jnp.float32)
    zdeg = jnp.zeros((ROWS_PER_TILE, DG), jnp.float32)
    psum, dsum = _sc_aggregate(x, edges, zrows, zdeg)
    return _tc_finish(
        x, psum, dsum, W_self, W_neigh,
        bias.reshape(1, D), ln_gamma.reshape(1, D), ln_beta.reshape(1, D),
    )
